# Initial kernel scaffold; baseline (speedup 1.0000x reference)
import jax
import jax.numpy as jnp
from jax.experimental import pallas as pl


def kernel(edge_index, x, W1, b1, W2, b2):
    raise NotImplementedError("write your pallas kernel here")


# speedup vs baseline: 16.8212x; 16.8212x over previous
import functools

import jax
import jax.numpy as jnp
from jax import lax
from jax.experimental import pallas as pl
from jax.experimental.pallas import tpu as pltpu
from jax.experimental.pallas import tpu_sc as plsc

N = 10000
E = 320000
D = 128
H = 16
C = 10

NC = 2
NS = 16
NW = NC * NS

NP = 10240
NB = 2 * NP
E_TOT = E + N
E_PAD = 330240
CH = E_PAD // NW
NCHUNK = CH // 16
COLS = NB // NS
ROWS_PER_TILE = NP // NS
ZROWS = 64

_f32 = jnp.float32
_i32 = jnp.int32


def _hist_body(src_hbm, dst_hbm, hist_hbm, src_v, dst_v, hist_v, red_v, acc_v,
               shared):
    cid = lax.axis_index("c")
    sid = lax.axis_index("s")
    wid = cid * NS + sid
    base = wid * CH
    pltpu.sync_copy(src_hbm.at[pl.ds(base, CH)], src_v)
    pltpu.sync_copy(dst_hbm.at[pl.ds(base, CH)], dst_v)

    def zero_body(i, _):
        hist_v[pl.ds(i * 16, 16)] = jnp.zeros((16,), _f32)
        return ()

    lax.fori_loop(0, NB // 16, zero_body, ())

    def edge_body(i, _):
        s16 = src_v[pl.ds(i * 16, 16)]
        d16 = dst_v[pl.ds(i * 16, 16)]
        cntd, lastd = plsc.scan_count(d16)
        plsc.addupdate_scatter(hist_v, [d16], cntd.astype(_f32), mask=lastd)
        m0 = d16 == 0
        cnts, lasts = plsc.scan_count(s16, mask=m0)
        plsc.addupdate_scatter(hist_v, [s16 + NP], cnts.astype(_f32),
                               mask=lasts)
        return ()

    lax.fori_loop(0, NCHUNK, edge_body, ())

    pltpu.sync_copy(hist_v, shared.at[sid])
    plsc.subcore_barrier()
    colbase = sid * COLS
    pltpu.sync_copy(shared.at[:, pl.ds(colbase, COLS)], red_v)

    def red_body(j, _):
        acc = red_v[0, pl.ds(j * 16, 16)]
        for t in range(1, NS):
            acc = acc + red_v[t, pl.ds(j * 16, 16)]
        acc_v[pl.ds(j * 16, 16)] = acc
        return ()

    lax.fori_loop(0, COLS // 16, red_body, ())
    pltpu.sync_copy(acc_v, hist_hbm.at[cid, pl.ds(colbase, COLS)])


def _make_hist_kernel():
    mesh = plsc.VectorSubcoreMesh(core_axis_name="c", subcore_axis_name="s")
    return pl.kernel(
        _hist_body,
        out_type=jax.ShapeDtypeStruct((NC, NB), _f32),
        mesh=mesh,
        compiler_params=pltpu.CompilerParams(needs_layout_passes=False),
        scratch_types=[
            pltpu.VMEM((CH,), _i32),
            pltpu.VMEM((CH,), _i32),
            pltpu.VMEM((NB,), _f32),
            pltpu.VMEM((NS, COLS), _f32),
            pltpu.VMEM((COLS,), _f32),
            pltpu.VMEM_SHARED((NS, NB), _f32),
        ],
    )


def _norm_body(hist_ref, dw_ref):
    h = hist_ref[0] + hist_ref[1]
    deg = h[: NP // 128]
    cnt0 = h[NP // 128:]
    dis = jnp.where(deg > 0.0, lax.rsqrt(jnp.maximum(deg, 1e-30)), 0.0)
    dis0 = dis[0:1, 0:1]
    w0 = cnt0 * dis * dis0
    dw_ref[0] = jnp.where(w0 > 0.0, -dis, dis)
    dw_ref[1] = w0


def _norm_kernel(hist):
    hist3 = hist.reshape(NC, NB // 128, 128)
    return pl.pallas_call(
        _norm_body,
        out_shape=jax.ShapeDtypeStruct((2, NP // 128, 128), _f32),
    )(hist3)


def _agg_body(src_hbm, dst_hbm, dism_hbm, x_hbm, agg_hbm,
              src_v, dst_v, dism_v, rows_v, nrm_v, zero_v, agg_sh):
    cid = lax.axis_index("c")
    sid = lax.axis_index("s")
    wid = cid * NS + sid
    base = wid * CH
    pltpu.sync_copy(src_hbm.at[pl.ds(base, CH)], src_v)
    pltpu.sync_copy(dst_hbm.at[pl.ds(base, CH)], dst_v)
    pltpu.sync_copy(dism_hbm, dism_v)

    def zb(r, _):
        for c in range(D // 16):
            zero_v[r, pl.ds(c * 16, 16)] = jnp.zeros((16,), _f32)
        return ()

    lax.fori_loop(0, ZROWS, zb, ())
    for k in range(ROWS_PER_TILE // ZROWS):
        pltpu.sync_copy(zero_v,
                        agg_sh.at[pl.ds(sid * ROWS_PER_TILE + k * ZROWS,
                                        ZROWS)])
    plsc.subcore_barrier()

    def edge_body(i, _):
        s16 = src_v[pl.ds(i * 16, 16)]
        d16 = dst_v[pl.ds(i * 16, 16)]
        dmd = plsc.load_gather(dism_v, [d16])
        m = dmd < 0.0

        @pl.when(jnp.any(m))
        def _():
            dms = plsc.load_gather(dism_v, [s16])
            nrm = jnp.where(m, jnp.abs(dms) * jnp.abs(dmd), 0.0)
            ssafe = jnp.where(m, s16, 0)
            dsafe = jnp.where(m, d16, 0)
            nrm_v[pl.ds(16, 16)] = nrm
            pltpu.sync_copy(x_hbm.at[ssafe], rows_v)
            for r in range(16):
                splat = plsc.load_gather(nrm_v,
                                         [jnp.full((16,), 16 + r, _i32)])
                for c in range(D // 16):
                    rows_v[r, pl.ds(c * 16, 16)] = (
                        rows_v[r, pl.ds(c * 16, 16)] * splat)
            pltpu.sync_copy(rows_v, agg_sh.at[dsafe], add=True)

        return ()

    lax.fori_loop(0, NCHUNK, edge_body, ())
    plsc.subcore_barrier()
    pltpu.sync_copy(agg_sh.at[pl.ds(sid * ROWS_PER_TILE, ROWS_PER_TILE)],
                    agg_hbm.at[cid, pl.ds(sid * ROWS_PER_TILE, ROWS_PER_TILE)])


def _make_agg_kernel():
    mesh = plsc.VectorSubcoreMesh(core_axis_name="c", subcore_axis_name="s")
    return pl.kernel(
        _agg_body,
        out_type=jax.ShapeDtypeStruct((NC, NP, D), _f32),
        mesh=mesh,
        compiler_params=pltpu.CompilerParams(needs_layout_passes=False),
        scratch_types=[
            pltpu.VMEM((CH,), _i32),
            pltpu.VMEM((CH,), _i32),
            pltpu.VMEM((NP,), _f32),
            pltpu.VMEM((16, D), _f32),
            pltpu.VMEM((32,), _f32),
            pltpu.VMEM((ZROWS, D), _f32),
            pltpu.VMEM_SHARED((NP, D), _f32),
        ],
    )


_BLK = 1024
_NSTEP = NP // _BLK


def _final_body(p0_ref, p1_ref, w0_ref, w1_ref, b1_ref, w2_ref, b2_ref,
                out_ref, acc_ref):
    i = pl.program_id(0)

    @pl.when(i == 0)
    def _():
        acc_ref[...] = jnp.zeros_like(acc_ref)

    a = p0_ref[...] + p1_ref[...]
    h = a @ w1_ref[...] + b1_ref[0:1, :]
    h = jnp.maximum(h, 0.0)
    acc_ref[...] += lax.dot_general(
        w0_ref[...], h, (((0,), (0,)), ((), ())),
        preferred_element_type=_f32)

    @pl.when(i == _NSTEP - 1)
    def _():
        z = jnp.dot(acc_ref[...], w2_ref[...],
                    preferred_element_type=_f32) + b2_ref[...]
        col = lax.broadcasted_iota(_i32, (8, 128), 1)
        zm = jnp.where(col < C, z, -jnp.inf)
        mx = jnp.max(zm, axis=1, keepdims=True)
        ez = jnp.where(col < C, jnp.exp(zm - mx), 0.0)
        se = jnp.sum(ez, axis=1, keepdims=True)
        out_ref[...] = zm - mx - jnp.log(se)


def _final_kernel(p0, p1, w0m, w1p, b1p, w2p, b2p):
    return pl.pallas_call(
        _final_body,
        grid=(_NSTEP,),
        in_specs=[
            pl.BlockSpec((_BLK, D), lambda i: (i, 0)),
            pl.BlockSpec((_BLK, D), lambda i: (i, 0)),
            pl.BlockSpec((_BLK, 8), lambda i: (i, 0)),
            pl.BlockSpec((D, D), lambda i: (0, 0)),
            pl.BlockSpec((8, D), lambda i: (0, 0)),
            pl.BlockSpec((D, D), lambda i: (0, 0)),
            pl.BlockSpec((8, D), lambda i: (0, 0)),
        ],
        out_specs=pl.BlockSpec((8, D), lambda i: (0, 0)),
        out_shape=jax.ShapeDtypeStruct((8, D), _f32),
        scratch_shapes=[pltpu.VMEM((8, D), _f32)],
    )(p0, p1, w0m, w1p, b1p, w2p, b2p)


def kernel(edge_index, x, W1, b1, W2, b2):
    loop = jnp.arange(N, dtype=_i32)
    pad = jnp.full((E_PAD - E_TOT,), N, dtype=_i32)
    src = jnp.concatenate([edge_index[0].astype(_i32), loop, pad])
    dst = jnp.concatenate([edge_index[1].astype(_i32), loop, pad])

    hist = _make_hist_kernel()(src, dst)
    dw = _norm_kernel(hist)
    dism = dw[0].reshape(NP)
    w0 = dw[1].reshape(NP)

    agg = _make_agg_kernel()(src, dst, dism, x)

    w0m = jnp.zeros((NP, 8), _f32).at[:, 0].set(w0)
    w1p = jnp.zeros((D, D), _f32).at[:, :H].set(W1)
    b1p = jnp.zeros((8, D), _f32).at[:, :H].set(jnp.broadcast_to(b1, (8, H)))
    w2p = jnp.zeros((D, D), _f32).at[:H, :C].set(W2)
    b2p = jnp.zeros((8, D), _f32).at[:, :C].set(jnp.broadcast_to(b2, (8, C)))

    res = _final_kernel(agg[0], agg[1], w0m, w1p, b1p, w2p, b2p)
    return res[0, :C]


# speedup vs baseline: 72.5462x; 4.3128x over previous
import functools

import jax
import jax.numpy as jnp
from jax import lax
from jax.experimental import pallas as pl
from jax.experimental.pallas import tpu as pltpu
from jax.experimental.pallas import tpu_sc as plsc

N = 10000
E = 320000
D = 128
H = 16
C = 10

NC = 2
NS = 16
NW = NC * NS

NP = 10240
NB = 2 * NP
E_TOT = E + N
E_PAD = 330240
CH = E_PAD // NW
NCHUNK = CH // 16
COLS = NB // NS
ROWS_PER_TILE = NP // NS
ZROWS = 64

_f32 = jnp.float32
_i32 = jnp.int32


def _hist_body(src_hbm, dst_hbm, hist_hbm, src_v, dst_v, hist_v, red_v, acc_v,
               shared):
    cid = lax.axis_index("c")
    sid = lax.axis_index("s")
    wid = cid * NS + sid
    base = wid * CH
    pltpu.sync_copy(src_hbm.at[pl.ds(base, CH)], src_v)
    pltpu.sync_copy(dst_hbm.at[pl.ds(base, CH)], dst_v)

    def zero_body(i, _):
        hist_v[pl.ds(i * 16, 16)] = jnp.zeros((16,), _f32)
        return ()

    lax.fori_loop(0, NB // 16, zero_body, ())

    def edge_body(i, _):
        s16 = src_v[pl.ds(i * 16, 16)]
        d16 = dst_v[pl.ds(i * 16, 16)]
        cntd, lastd = plsc.scan_count(d16)
        plsc.addupdate_scatter(hist_v, [d16], cntd.astype(_f32), mask=lastd)
        m0 = d16 == 0
        cnts, lasts = plsc.scan_count(s16, mask=m0)
        plsc.addupdate_scatter(hist_v, [s16 + NP], cnts.astype(_f32),
                               mask=lasts)
        return ()

    lax.fori_loop(0, NCHUNK, edge_body, ())

    pltpu.sync_copy(hist_v, shared.at[sid])
    plsc.subcore_barrier()
    colbase = sid * COLS
    pltpu.sync_copy(shared.at[:, pl.ds(colbase, COLS)], red_v)

    def red_body(j, _):
        acc = red_v[0, pl.ds(j * 16, 16)]
        for t in range(1, NS):
            acc = acc + red_v[t, pl.ds(j * 16, 16)]
        acc_v[pl.ds(j * 16, 16)] = acc
        return ()

    lax.fori_loop(0, COLS // 16, red_body, ())
    pltpu.sync_copy(acc_v, hist_hbm.at[cid, pl.ds(colbase, COLS)])


def _make_hist_kernel():
    mesh = plsc.VectorSubcoreMesh(core_axis_name="c", subcore_axis_name="s")
    return pl.kernel(
        _hist_body,
        out_type=jax.ShapeDtypeStruct((NC, NB), _f32),
        mesh=mesh,
        compiler_params=pltpu.CompilerParams(needs_layout_passes=False),
        scratch_types=[
            pltpu.VMEM((CH,), _i32),
            pltpu.VMEM((CH,), _i32),
            pltpu.VMEM((NB,), _f32),
            pltpu.VMEM((NS, COLS), _f32),
            pltpu.VMEM((COLS,), _f32),
            pltpu.VMEM_SHARED((NS, NB), _f32),
        ],
    )


def _norm_body(hist_ref, dw_ref):
    h = hist_ref[0] + hist_ref[1]
    deg = h[: NP // 128]
    cnt0 = h[NP // 128:]
    dis = jnp.where(deg > 0.0, lax.rsqrt(jnp.maximum(deg, 1e-30)), 0.0)
    dis0 = dis[0:1, 0:1]
    w0 = cnt0 * dis * dis0
    dw_ref[0] = jnp.where(w0 > 0.0, -dis, dis)
    dw_ref[1] = w0


def _norm_kernel(hist):
    hist3 = hist.reshape(NC, NB // 128, 128)
    return pl.pallas_call(
        _norm_body,
        out_shape=jax.ShapeDtypeStruct((2, NP // 128, 128), _f32),
    )(hist3)


def _agg_body(src_hbm, dst_hbm, dism_hbm, x_hbm, agg_hbm,
              src_v, dst_v, dism_v, rows_v, nrm_v, zero_v, agg_sh):
    cid = lax.axis_index("c")
    sid = lax.axis_index("s")
    wid = cid * NS + sid
    base = wid * CH

    def zb(r, _):
        for c in range(D // 16):
            zero_v[r, pl.ds(c * 16, 16)] = jnp.zeros((16,), _f32)
        return ()

    lax.fori_loop(0, ZROWS, zb, ())
    pltpu.sync_copy(src_hbm.at[pl.ds(base, CH)], src_v.at[pl.ds(0, CH)])
    pltpu.sync_copy(dst_hbm.at[pl.ds(base, CH)], dst_v.at[pl.ds(0, CH)])
    pltpu.sync_copy(dism_hbm, dism_v)
    for k in range(ROWS_PER_TILE // ZROWS):
        pltpu.sync_copy(
            zero_v, agg_sh.at[pl.ds(sid * ROWS_PER_TILE + k * ZROWS, ZROWS)])
    plsc.subcore_barrier()

    def edge_body(i, cnt):
        s16 = src_v[pl.ds(i * 16, 16)]
        d16 = dst_v[pl.ds(i * 16, 16)]
        dmd = plsc.load_gather(dism_v, [d16])
        m = dmd < 0.0
        idx = cnt + plsc.cumsum(m.astype(_i32)) - 1
        plsc.store_scatter(src_v, [idx], s16, mask=m)
        plsc.store_scatter(dst_v, [idx], d16, mask=m)
        return cnt + jnp.sum(m.astype(_i32))

    cnt = lax.fori_loop(0, NCHUNK, edge_body, jnp.int32(0))

    tail = cnt + lax.iota(_i32, 16)
    plsc.store_scatter(src_v, [tail], jnp.zeros((16,), _i32))
    plsc.store_scatter(dst_v, [tail], jnp.full((16,), N, _i32))

    def proc_body(j, _):
        s16 = src_v[pl.ds(j * 16, 16)]
        d16 = dst_v[pl.ds(j * 16, 16)]
        dms = plsc.load_gather(dism_v, [s16])
        dmd = plsc.load_gather(dism_v, [d16])
        nrm = jnp.abs(dms) * jnp.abs(dmd)
        nrm_v[pl.ds(16, 16)] = nrm
        pltpu.sync_copy(x_hbm.at[s16], rows_v)
        for r in range(16):
            splat = plsc.load_gather(nrm_v, [jnp.full((16,), 16 + r, _i32)])
            for c in range(D // 16):
                rows_v[r, pl.ds(c * 16, 16)] = (
                    rows_v[r, pl.ds(c * 16, 16)] * splat)
        pltpu.sync_copy(rows_v, agg_sh.at[d16], add=True)
        return ()

    nch = (cnt + 15) // 16
    lax.fori_loop(0, nch, proc_body, ())
    plsc.subcore_barrier()
    pltpu.sync_copy(agg_sh.at[pl.ds(sid * ROWS_PER_TILE, ROWS_PER_TILE)],
                    agg_hbm.at[cid, pl.ds(sid * ROWS_PER_TILE, ROWS_PER_TILE)])


def _make_agg_kernel():
    mesh = plsc.VectorSubcoreMesh(core_axis_name="c", subcore_axis_name="s")
    return pl.kernel(
        _agg_body,
        out_type=jax.ShapeDtypeStruct((NC, NP, D), _f32),
        mesh=mesh,
        compiler_params=pltpu.CompilerParams(needs_layout_passes=False),
        scratch_types=[
            pltpu.VMEM((CH + 16,), _i32),
            pltpu.VMEM((CH + 16,), _i32),
            pltpu.VMEM((NP,), _f32),
            pltpu.VMEM((16, D), _f32),
            pltpu.VMEM((32,), _f32),
            pltpu.VMEM((ZROWS, D), _f32),
            pltpu.VMEM_SHARED((NP, D), _f32),
        ],
    )


_BLK = 1024
_NSTEP = NP // _BLK


def _final_body(p0_ref, p1_ref, w0_ref, w1_ref, b1_ref, w2_ref, b2_ref,
                out_ref, acc_ref):
    i = pl.program_id(0)

    @pl.when(i == 0)
    def _():
        acc_ref[...] = jnp.zeros_like(acc_ref)

    a = p0_ref[...] + p1_ref[...]
    h = a @ w1_ref[...] + b1_ref[0:1, :]
    h = jnp.maximum(h, 0.0)
    acc_ref[...] += lax.dot_general(
        w0_ref[...], h, (((0,), (0,)), ((), ())),
        preferred_element_type=_f32)

    @pl.when(i == _NSTEP - 1)
    def _():
        z = jnp.dot(acc_ref[...], w2_ref[...],
                    preferred_element_type=_f32) + b2_ref[...]
        col = lax.broadcasted_iota(_i32, (8, 128), 1)
        zm = jnp.where(col < C, z, -jnp.inf)
        mx = jnp.max(zm, axis=1, keepdims=True)
        ez = jnp.where(col < C, jnp.exp(zm - mx), 0.0)
        se = jnp.sum(ez, axis=1, keepdims=True)
        out_ref[...] = zm - mx - jnp.log(se)


def _final_kernel(p0, p1, w0m, w1p, b1p, w2p, b2p):
    return pl.pallas_call(
        _final_body,
        grid=(_NSTEP,),
        in_specs=[
            pl.BlockSpec((_BLK, D), lambda i: (i, 0)),
            pl.BlockSpec((_BLK, D), lambda i: (i, 0)),
            pl.BlockSpec((_BLK, 8), lambda i: (i, 0)),
            pl.BlockSpec((D, D), lambda i: (0, 0)),
            pl.BlockSpec((8, D), lambda i: (0, 0)),
            pl.BlockSpec((D, D), lambda i: (0, 0)),
            pl.BlockSpec((8, D), lambda i: (0, 0)),
        ],
        out_specs=pl.BlockSpec((8, D), lambda i: (0, 0)),
        out_shape=jax.ShapeDtypeStruct((8, D), _f32),
        scratch_shapes=[pltpu.VMEM((8, D), _f32)],
    )(p0, p1, w0m, w1p, b1p, w2p, b2p)


def kernel(edge_index, x, W1, b1, W2, b2):
    loop = jnp.arange(N, dtype=_i32)
    pad = jnp.full((E_PAD - E_TOT,), N, dtype=_i32)
    src = jnp.concatenate([edge_index[0].astype(_i32), loop, pad])
    dst = jnp.concatenate([edge_index[1].astype(_i32), loop, pad])

    hist = _make_hist_kernel()(src, dst)
    dw = _norm_kernel(hist)
    dism = dw[0].reshape(NP)
    w0 = dw[1].reshape(NP)

    agg = _make_agg_kernel()(src, dst, dism, x)

    w0m = jnp.zeros((NP, 8), _f32).at[:, 0].set(w0)
    w1p = jnp.zeros((D, D), _f32).at[:, :H].set(W1)
    b1p = jnp.zeros((8, D), _f32).at[:, :H].set(jnp.broadcast_to(b1, (8, H)))
    w2p = jnp.zeros((D, D), _f32).at[:H, :C].set(W2)
    b2p = jnp.zeros((8, D), _f32).at[:, :C].set(jnp.broadcast_to(b2, (8, C)))

    res = _final_kernel(agg[0], agg[1], w0m, w1p, b1p, w2p, b2p)
    return res[0, :C]


# speedup vs baseline: 90.5896x; 1.2487x over previous
import functools

import jax
import jax.numpy as jnp
from jax import lax
from jax.experimental import pallas as pl
from jax.experimental.pallas import tpu as pltpu
from jax.experimental.pallas import tpu_sc as plsc

N = 10000
E = 320000
D = 128
H = 16
C = 10

NC = 2
NS = 16
NW = NC * NS

NP = 10240
NB = 2 * NP
CHE = E // NW
NCHUNK_E = CHE // 16
NLOOP = NP // NW
NCHUNK_S = NLOOP // 16
CAP = CHE + NLOOP + 16
COLS = NB // NS
ROWS_PER_TILE = NP // NS

_f32 = jnp.float32
_i32 = jnp.int32


def _hist_body(ei_hbm, hist_hbm, src_v, dst_v, hist_v, red_v, acc_v, shared):
    cid = lax.axis_index("c")
    sid = lax.axis_index("s")
    wid = cid * NS + sid
    base = wid * CHE
    pltpu.sync_copy(ei_hbm.at[pl.ds(base, CHE)], src_v)
    pltpu.sync_copy(ei_hbm.at[pl.ds(E + base, CHE)], dst_v)

    def zero_body(i, _):
        hist_v[pl.ds(i * 16, 16)] = jnp.zeros((16,), _f32)
        return ()

    lax.fori_loop(0, NB // 16, zero_body, ())

    ones = jnp.ones((16,), _f32)

    def edge_body(i, _):
        s16 = src_v[pl.ds(i * 16, 16)]
        d16 = dst_v[pl.ds(i * 16, 16)]
        plsc.addupdate_scatter(hist_v, [d16], ones)
        plsc.addupdate_scatter(hist_v, [s16 + NP], ones, mask=d16 == 0)
        return ()

    lax.fori_loop(0, NCHUNK_E, edge_body, ())

    pltpu.sync_copy(hist_v, shared.at[sid])
    plsc.subcore_barrier()
    colbase = sid * COLS
    pltpu.sync_copy(shared.at[:, pl.ds(colbase, COLS)], red_v)

    def red_body(j, _):
        acc = red_v[0, pl.ds(j * 16, 16)]
        for t in range(1, NS):
            acc = acc + red_v[t, pl.ds(j * 16, 16)]
        acc_v[pl.ds(j * 16, 16)] = acc
        return ()

    lax.fori_loop(0, COLS // 16, red_body, ())
    pltpu.sync_copy(acc_v, hist_hbm.at[cid, pl.ds(colbase, COLS)])


def _make_hist_kernel():
    mesh = plsc.VectorSubcoreMesh(core_axis_name="c", subcore_axis_name="s")
    return pl.kernel(
        _hist_body,
        out_type=jax.ShapeDtypeStruct((NC, NB), _f32),
        mesh=mesh,
        compiler_params=pltpu.CompilerParams(needs_layout_passes=False),
        scratch_types=[
            pltpu.VMEM((CHE,), _i32),
            pltpu.VMEM((CHE,), _i32),
            pltpu.VMEM((NB,), _f32),
            pltpu.VMEM((NS, COLS), _f32),
            pltpu.VMEM((COLS,), _f32),
            pltpu.VMEM_SHARED((NS, NB), _f32),
        ],
    )


def _norm_body(hist_ref, dw_ref):
    h = hist_ref[0] + hist_ref[1]
    row = lax.broadcasted_iota(_i32, (NP // 128, 128), 0)
    col = lax.broadcasted_iota(_i32, (NP // 128, 128), 1)
    at0 = jnp.logical_and(row == 0, col == 0).astype(_f32)
    deg = h[: NP // 128] + 1.0
    cnt0 = h[NP // 128:] + at0
    dis = lax.rsqrt(deg)
    dis0 = dis[0:1, 0:1]
    w0 = cnt0 * dis * dis0
    dw_ref[0] = jnp.where(w0 > 0.0, -dis, dis)
    dw_ref[1] = w0


def _norm_kernel(hist):
    hist3 = hist.reshape(NC, NB // 128, 128)
    return pl.pallas_call(
        _norm_body,
        out_shape=jax.ShapeDtypeStruct((2, NP // 128, 128), _f32),
    )(hist3)


def _agg_body(ei_hbm, dism_hbm, x_hbm, zeros_hbm, agg_hbm,
              src_v, dst_v, dism_v, rows_v, nrm_v, agg_sh):
    cid = lax.axis_index("c")
    sid = lax.axis_index("s")
    wid = cid * NS + sid
    base = wid * CHE
    pltpu.sync_copy(ei_hbm.at[pl.ds(base, CHE)], src_v.at[pl.ds(0, CHE)])
    pltpu.sync_copy(ei_hbm.at[pl.ds(E + base, CHE)],
                    dst_v.at[pl.ds(0, CHE)])
    pltpu.sync_copy(dism_hbm, dism_v)
    pltpu.sync_copy(zeros_hbm.at[pl.ds(sid * ROWS_PER_TILE, ROWS_PER_TILE)],
                    agg_sh.at[pl.ds(sid * ROWS_PER_TILE, ROWS_PER_TILE)])
    plsc.subcore_barrier()

    def compact(cntv, s16, d16, m):
        idx = cntv + plsc.cumsum(m.astype(_i32)) - 1
        plsc.store_scatter(src_v, [idx], s16, mask=m)
        plsc.store_scatter(dst_v, [idx], d16, mask=m)
        return cntv + plsc.all_reduce_population_count(m)

    def edge_body(i, cntv):
        s16 = src_v[pl.ds(i * 16, 16)]
        d16 = dst_v[pl.ds(i * 16, 16)]
        m = plsc.load_gather(dism_v, [d16]) < 0.0
        return lax.cond(jnp.any(m), lambda: compact(cntv, s16, d16, m),
                        lambda: cntv)

    cntv = lax.fori_loop(0, NCHUNK_E, edge_body, jnp.zeros((16,), _i32))

    nbase = wid * NLOOP

    def loop_body(i, cntv):
        v16 = nbase + i * 16 + lax.iota(_i32, 16)
        m = plsc.load_gather(dism_v, [v16]) < 0.0
        return lax.cond(jnp.any(m), lambda: compact(cntv, v16, v16, m),
                        lambda: cntv)

    cntv = lax.fori_loop(0, NCHUNK_S, loop_body, cntv)
    cnt = jnp.max(cntv)

    tail = cnt + lax.iota(_i32, 16)
    plsc.store_scatter(src_v, [tail], jnp.zeros((16,), _i32))
    plsc.store_scatter(dst_v, [tail], jnp.full((16,), N, _i32))

    def proc_body(j, _):
        s16 = src_v[pl.ds(j * 16, 16)]
        d16 = dst_v[pl.ds(j * 16, 16)]
        dms = plsc.load_gather(dism_v, [s16])
        dmd = plsc.load_gather(dism_v, [d16])
        nrm = jnp.abs(dms) * jnp.abs(dmd)
        nrm_v[pl.ds(16, 16)] = nrm
        pltpu.sync_copy(x_hbm.at[s16], rows_v)
        for r in range(16):
            splat = plsc.load_gather(nrm_v, [jnp.full((16,), 16 + r, _i32)])
            for c in range(D // 16):
                rows_v[r, pl.ds(c * 16, 16)] = (
                    rows_v[r, pl.ds(c * 16, 16)] * splat)
        pltpu.sync_copy(rows_v, agg_sh.at[d16], add=True)
        return ()

    nch = (cnt + 15) // 16
    lax.fori_loop(0, nch, proc_body, ())
    plsc.subcore_barrier()
    pltpu.sync_copy(agg_sh.at[pl.ds(sid * ROWS_PER_TILE, ROWS_PER_TILE)],
                    agg_hbm.at[cid, pl.ds(sid * ROWS_PER_TILE, ROWS_PER_TILE)])


def _make_agg_kernel():
    mesh = plsc.VectorSubcoreMesh(core_axis_name="c", subcore_axis_name="s")
    return pl.kernel(
        _agg_body,
        out_type=jax.ShapeDtypeStruct((NC, NP, D), _f32),
        mesh=mesh,
        compiler_params=pltpu.CompilerParams(needs_layout_passes=False),
        scratch_types=[
            pltpu.VMEM((CAP,), _i32),
            pltpu.VMEM((CAP,), _i32),
            pltpu.VMEM((NP,), _f32),
            pltpu.VMEM((16, D), _f32),
            pltpu.VMEM((32,), _f32),
            pltpu.VMEM_SHARED((NP, D), _f32),
        ],
    )


_BLK = 1024
_NSTEP = NP // _BLK


def _final_body(p0_ref, p1_ref, w0_ref, w1_ref, b1_ref, w2_ref, b2_ref,
                out_ref, acc_ref):
    i = pl.program_id(0)

    @pl.when(i == 0)
    def _():
        acc_ref[...] = jnp.zeros_like(acc_ref)

    a = p0_ref[0] + p1_ref[0]
    h = a @ w1_ref[...] + b1_ref[...]
    h = jnp.maximum(h, 0.0)
    acc_ref[...] += lax.dot_general(
        w0_ref[...], h, (((0,), (0,)), ((), ())),
        preferred_element_type=_f32)

    @pl.when(i == _NSTEP - 1)
    def _():
        z = jnp.dot(acc_ref[...], w2_ref[...],
                    preferred_element_type=_f32) + b2_ref[...]
        mx = jnp.max(z, axis=1, keepdims=True)
        se = jnp.sum(jnp.exp(z - mx), axis=1, keepdims=True)
        out_ref[...] = z - mx - jnp.log(se)


def _final_kernel(agg, w0m, W1, b1, W2, b2):
    return pl.pallas_call(
        _final_body,
        grid=(_NSTEP,),
        in_specs=[
            pl.BlockSpec((1, _BLK, D), lambda i: (0, i, 0)),
            pl.BlockSpec((1, _BLK, D), lambda i: (1, i, 0)),
            pl.BlockSpec((_BLK, 8), lambda i: (i, 0)),
            pl.BlockSpec((D, H), lambda i: (0, 0)),
            pl.BlockSpec((1, H), lambda i: (0, 0)),
            pl.BlockSpec((H, C), lambda i: (0, 0)),
            pl.BlockSpec((1, C), lambda i: (0, 0)),
        ],
        out_specs=pl.BlockSpec((8, C), lambda i: (0, 0)),
        out_shape=jax.ShapeDtypeStruct((8, C), _f32),
        scratch_shapes=[pltpu.VMEM((8, H), _f32)],
    )(agg, agg, w0m, W1, b1, W2, b2)


def kernel(edge_index, x, W1, b1, W2, b2):
    ei = edge_index.astype(_i32).reshape(2 * E)
    hist = _make_hist_kernel()(ei)
    dw = _norm_kernel(hist)
    dism = dw[0].reshape(NP)
    w0 = dw[1].reshape(NP)

    zeros_big = jnp.zeros((NP, D), _f32)
    agg = _make_agg_kernel()(ei, dism, x, zeros_big)

    w0m = jnp.zeros((NP, 8), _f32).at[:, 0].set(w0)
    res = _final_kernel(agg, w0m, W1, b1.reshape(1, H), W2, b2.reshape(1, C))
    return res[0]


# speedup vs baseline: 93.8883x; 1.0364x over previous
import functools

import jax
import jax.numpy as jnp
from jax import lax
from jax.experimental import pallas as pl
from jax.experimental.pallas import tpu as pltpu
from jax.experimental.pallas import tpu_sc as plsc

N = 10000
E = 320000
D = 128
H = 16
C = 10

NC = 2
NS = 16
NW = NC * NS

NP = 10240
NB = 2 * NP
CHE = E // NW
NCHUNK_E = CHE // 16
NLOOP = NP // NW
NCHUNK_S = NLOOP // 16
CAP = CHE + NLOOP + 16
COLS = NB // NS
ROWS_PER_TILE = NP // NS
ZROWS = 64

_f32 = jnp.float32
_i32 = jnp.int32


def _hist_body(ei_hbm, hist_hbm, src_v, dst_v, hist_v, red_v, acc_v, shared):
    cid = lax.axis_index("c")
    sid = lax.axis_index("s")
    wid = cid * NS + sid
    base = wid * CHE
    pltpu.sync_copy(ei_hbm.at[pl.ds(base, CHE)], src_v)
    pltpu.sync_copy(ei_hbm.at[pl.ds(E + base, CHE)], dst_v)

    def zero_body(i, _):
        hist_v[pl.ds(i * 16, 16)] = jnp.zeros((16,), _f32)
        return ()

    lax.fori_loop(0, NB // 16, zero_body, ())

    ones = jnp.ones((16,), _f32)

    def edge_body(i, _):
        s16 = src_v[pl.ds(i * 16, 16)]
        d16 = dst_v[pl.ds(i * 16, 16)]
        plsc.addupdate_scatter(hist_v, [d16], ones)
        plsc.addupdate_scatter(hist_v, [s16 + NP], ones, mask=d16 == 0)
        return ()

    lax.fori_loop(0, NCHUNK_E, edge_body, ())

    pltpu.sync_copy(hist_v, shared.at[sid])
    plsc.subcore_barrier()
    colbase = sid * COLS
    pltpu.sync_copy(shared.at[:, pl.ds(colbase, COLS)], red_v)

    def red_body(j, _):
        acc = red_v[0, pl.ds(j * 16, 16)]
        for t in range(1, NS):
            acc = acc + red_v[t, pl.ds(j * 16, 16)]
        acc_v[pl.ds(j * 16, 16)] = acc
        return ()

    lax.fori_loop(0, COLS // 16, red_body, ())
    pltpu.sync_copy(acc_v, hist_hbm.at[cid, pl.ds(colbase, COLS)])


def _make_hist_kernel():
    mesh = plsc.VectorSubcoreMesh(core_axis_name="c", subcore_axis_name="s")
    return pl.kernel(
        _hist_body,
        out_type=jax.ShapeDtypeStruct((NC, NB), _f32),
        mesh=mesh,
        compiler_params=pltpu.CompilerParams(needs_layout_passes=False),
        scratch_types=[
            pltpu.VMEM((CHE,), _i32),
            pltpu.VMEM((CHE,), _i32),
            pltpu.VMEM((NB,), _f32),
            pltpu.VMEM((NS, COLS), _f32),
            pltpu.VMEM((COLS,), _f32),
            pltpu.VMEM_SHARED((NS, NB), _f32),
        ],
    )


def _norm_body(hist_ref, dw_ref):
    h = hist_ref[0] + hist_ref[1]
    row = lax.broadcasted_iota(_i32, (NP // 128, 128), 0)
    col = lax.broadcasted_iota(_i32, (NP // 128, 128), 1)
    at0 = jnp.logical_and(row == 0, col == 0).astype(_f32)
    deg = h[: NP // 128] + 1.0
    cnt0 = h[NP // 128:] + at0
    dis = lax.rsqrt(deg)
    dis0 = dis[0:1, 0:1]
    w0 = cnt0 * dis * dis0
    dw_ref[0] = jnp.where(w0 > 0.0, -dis, dis)
    dw_ref[1] = w0


def _norm_kernel(hist):
    hist3 = hist.reshape(NC, NB // 128, 128)
    return pl.pallas_call(
        _norm_body,
        out_shape=jax.ShapeDtypeStruct((2, NP // 128, 128), _f32),
    )(hist3)


def _agg_body(ei_hbm, dism_hbm, x_hbm, agg_hbm,
              src_v, dst_v, dism_v, rows_v, nrm_v, zero_v, agg_sh):
    cid = lax.axis_index("c")
    sid = lax.axis_index("s")
    wid = cid * NS + sid
    base = wid * CHE
    pltpu.sync_copy(ei_hbm.at[pl.ds(base, CHE)], src_v.at[pl.ds(0, CHE)])
    pltpu.sync_copy(ei_hbm.at[pl.ds(E + base, CHE)],
                    dst_v.at[pl.ds(0, CHE)])
    pltpu.sync_copy(dism_hbm, dism_v)

    def zb(r, _):
        for c in range(D // 16):
            zero_v[r, pl.ds(c * 16, 16)] = jnp.zeros((16,), _f32)
        return ()

    lax.fori_loop(0, ZROWS, zb, ())
    for k in range(ROWS_PER_TILE // ZROWS):
        pltpu.sync_copy(
            zero_v, agg_sh.at[pl.ds(sid * ROWS_PER_TILE + k * ZROWS, ZROWS)])
    plsc.subcore_barrier()

    def compact(cntv, s16, d16, m):
        idx = cntv + plsc.cumsum(m.astype(_i32)) - 1
        plsc.store_scatter(src_v, [idx], s16, mask=m)
        plsc.store_scatter(dst_v, [idx], d16, mask=m)
        return cntv + plsc.all_reduce_population_count(m)

    def edge_body(i, cntv):
        s16 = src_v[pl.ds(i * 16, 16)]
        d16 = dst_v[pl.ds(i * 16, 16)]
        m = plsc.load_gather(dism_v, [d16]) < 0.0
        return lax.cond(jnp.any(m), lambda: compact(cntv, s16, d16, m),
                        lambda: cntv)

    cntv = lax.fori_loop(0, NCHUNK_E, edge_body, jnp.zeros((16,), _i32))

    nbase = wid * NLOOP

    def loop_body(i, cntv):
        v16 = nbase + i * 16 + lax.iota(_i32, 16)
        m = plsc.load_gather(dism_v, [v16]) < 0.0
        return lax.cond(jnp.any(m), lambda: compact(cntv, v16, v16, m),
                        lambda: cntv)

    cntv = lax.fori_loop(0, NCHUNK_S, loop_body, cntv)
    cnt = jnp.max(cntv)

    tail = cnt + lax.iota(_i32, 16)
    plsc.store_scatter(src_v, [tail], jnp.zeros((16,), _i32))
    plsc.store_scatter(dst_v, [tail], jnp.full((16,), N, _i32))

    def proc_body(j, _):
        s16 = src_v[pl.ds(j * 16, 16)]
        d16 = dst_v[pl.ds(j * 16, 16)]
        dms = plsc.load_gather(dism_v, [s16])
        dmd = plsc.load_gather(dism_v, [d16])
        nrm = jnp.abs(dms) * jnp.abs(dmd)
        nrm_v[pl.ds(16, 16)] = nrm
        pltpu.sync_copy(x_hbm.at[s16], rows_v)
        for r in range(16):
            splat = plsc.load_gather(nrm_v, [jnp.full((16,), 16 + r, _i32)])
            for c in range(D // 16):
                rows_v[r, pl.ds(c * 16, 16)] = (
                    rows_v[r, pl.ds(c * 16, 16)] * splat)
        pltpu.sync_copy(rows_v, agg_sh.at[d16], add=True)
        return ()

    nch = (cnt + 15) // 16
    lax.fori_loop(0, nch, proc_body, ())
    plsc.subcore_barrier()
    pltpu.sync_copy(agg_sh.at[pl.ds(sid * ROWS_PER_TILE, ROWS_PER_TILE)],
                    agg_hbm.at[cid, pl.ds(sid * ROWS_PER_TILE, ROWS_PER_TILE)])


def _make_agg_kernel():
    mesh = plsc.VectorSubcoreMesh(core_axis_name="c", subcore_axis_name="s")
    return pl.kernel(
        _agg_body,
        out_type=jax.ShapeDtypeStruct((NC, NP, D), _f32),
        mesh=mesh,
        compiler_params=pltpu.CompilerParams(needs_layout_passes=False),
        scratch_types=[
            pltpu.VMEM((CAP,), _i32),
            pltpu.VMEM((CAP,), _i32),
            pltpu.VMEM((NP,), _f32),
            pltpu.VMEM((16, D), _f32),
            pltpu.VMEM((32,), _f32),
            pltpu.VMEM((ZROWS, D), _f32),
            pltpu.VMEM_SHARED((NP, D), _f32),
        ],
    )


_BLK = 1024
_NSTEP = NP // _BLK


def _final_body(p0_ref, p1_ref, w0_ref, w1_ref, b1_ref, w2_ref, b2_ref,
                out_ref, acc_ref):
    i = pl.program_id(0)

    @pl.when(i == 0)
    def _():
        acc_ref[...] = jnp.zeros_like(acc_ref)

    a = p0_ref[0] + p1_ref[0]
    h = a @ w1_ref[...] + b1_ref[...]
    h = jnp.maximum(h, 0.0)
    acc_ref[...] += lax.dot_general(
        w0_ref[...], h, (((0,), (0,)), ((), ())),
        preferred_element_type=_f32)

    @pl.when(i == _NSTEP - 1)
    def _():
        z = jnp.dot(acc_ref[...], w2_ref[...],
                    preferred_element_type=_f32) + b2_ref[...]
        mx = jnp.max(z, axis=1, keepdims=True)
        se = jnp.sum(jnp.exp(z - mx), axis=1, keepdims=True)
        out_ref[...] = z - mx - jnp.log(se)


def _final_kernel(agg, w0m, W1, b1, W2, b2):
    return pl.pallas_call(
        _final_body,
        grid=(_NSTEP,),
        in_specs=[
            pl.BlockSpec((1, _BLK, D), lambda i: (0, i, 0)),
            pl.BlockSpec((1, _BLK, D), lambda i: (1, i, 0)),
            pl.BlockSpec((_BLK, 8), lambda i: (i, 0)),
            pl.BlockSpec((D, H), lambda i: (0, 0)),
            pl.BlockSpec((1, H), lambda i: (0, 0)),
            pl.BlockSpec((H, C), lambda i: (0, 0)),
            pl.BlockSpec((1, C), lambda i: (0, 0)),
        ],
        out_specs=pl.BlockSpec((8, C), lambda i: (0, 0)),
        out_shape=jax.ShapeDtypeStruct((8, C), _f32),
        scratch_shapes=[pltpu.VMEM((8, H), _f32)],
    )(agg, agg, w0m, W1, b1, W2, b2)


def kernel(edge_index, x, W1, b1, W2, b2):
    ei = edge_index.astype(_i32).reshape(2 * E)
    hist = _make_hist_kernel()(ei)
    dw = _norm_kernel(hist)
    dism = dw[0].reshape(NP)
    w0 = dw[1].reshape(NP)

    agg = _make_agg_kernel()(ei, dism, x)

    w0m = jnp.zeros((NP, 8), _f32).at[:, 0].set(w0)
    res = _final_kernel(agg, w0m, W1, b1.reshape(1, H), W2, b2.reshape(1, C))
    return res[0]


# speedup vs baseline: 103.2823x; 1.1001x over previous
import functools

import jax
import jax.numpy as jnp
from jax import lax
from jax.experimental import pallas as pl
from jax.experimental.pallas import tpu as pltpu
from jax.experimental.pallas import tpu_sc as plsc

N = 10000
E = 320000
D = 128
H = 16
C = 10

NC = 2
NS = 16
NW = NC * NS

NP = 10240
NB = 2 * NP
CHE = E // NW
NCHUNK_E = CHE // 16
NLOOP = NP // NW
NCHUNK_S = NLOOP // 16
CAP = CHE + NLOOP + 16
COLS = NB // NS
ROWS_PER_TILE = NP // NS
ZROWS = 64

_f32 = jnp.float32
_i32 = jnp.int32


def _hist_body(ei_hbm, hist_hbm, src_v, dst_v, hist_v, red_v, acc_v, shared):
    cid = lax.axis_index("c")
    sid = lax.axis_index("s")
    wid = cid * NS + sid
    base = wid * CHE
    pltpu.sync_copy(ei_hbm.at[pl.ds(base, CHE)], src_v)
    pltpu.sync_copy(ei_hbm.at[pl.ds(E + base, CHE)], dst_v)

    def zero_body(i, _):
        hist_v[pl.ds(i * 16, 16)] = jnp.zeros((16,), _f32)
        return ()

    lax.fori_loop(0, NB // 16, zero_body, ())

    ones = jnp.ones((16,), _f32)

    def edge_body(i, _):
        s16 = src_v[pl.ds(i * 16, 16)]
        d16 = dst_v[pl.ds(i * 16, 16)]
        plsc.addupdate_scatter(hist_v, [d16], ones)
        plsc.addupdate_scatter(hist_v, [s16 + NP], ones, mask=d16 == 0)
        return ()

    lax.fori_loop(0, NCHUNK_E, edge_body, ())

    pltpu.sync_copy(hist_v, shared.at[sid])
    plsc.subcore_barrier()
    colbase = sid * COLS
    pltpu.sync_copy(shared.at[:, pl.ds(colbase, COLS)], red_v)

    def red_body(j, _):
        acc = red_v[0, pl.ds(j * 16, 16)]
        for t in range(1, NS):
            acc = acc + red_v[t, pl.ds(j * 16, 16)]
        acc_v[pl.ds(j * 16, 16)] = acc
        return ()

    lax.fori_loop(0, COLS // 16, red_body, ())
    pltpu.sync_copy(acc_v, hist_hbm.at[cid, pl.ds(colbase, COLS)])


def _make_hist_kernel():
    mesh = plsc.VectorSubcoreMesh(core_axis_name="c", subcore_axis_name="s")
    return pl.kernel(
        _hist_body,
        out_type=jax.ShapeDtypeStruct((NC, NB), _f32),
        mesh=mesh,
        compiler_params=pltpu.CompilerParams(needs_layout_passes=False),
        scratch_types=[
            pltpu.VMEM((CHE,), _i32),
            pltpu.VMEM((CHE,), _i32),
            pltpu.VMEM((NB,), _f32),
            pltpu.VMEM((NS, COLS), _f32),
            pltpu.VMEM((COLS,), _f32),
            pltpu.VMEM_SHARED((NS, NB), _f32),
        ],
    )


def _norm_body(hist_ref, dw_ref):
    h = hist_ref[0] + hist_ref[1]
    row = lax.broadcasted_iota(_i32, (NP // 128, 128), 0)
    col = lax.broadcasted_iota(_i32, (NP // 128, 128), 1)
    at0 = jnp.logical_and(row == 0, col == 0).astype(_f32)
    deg = h[: NP // 128] + 1.0
    cnt0 = h[NP // 128:] + at0
    dis = lax.rsqrt(deg)
    dis0 = dis[0:1, 0:1]
    w0 = cnt0 * dis * dis0
    dw_ref[0] = jnp.where(w0 > 0.0, -dis, dis)
    dw_ref[1] = w0


def _norm_kernel(hist):
    hist3 = hist.reshape(NC, NB // 128, 128)
    return pl.pallas_call(
        _norm_body,
        out_shape=jax.ShapeDtypeStruct((2, NP // 128, 128), _f32),
    )(hist3)


def _agg_body(ei_hbm, dism_hbm, x_hbm, agg_hbm,
              src_v, dst_v, dism_v, rows_v, nrm_v, zero_v, agg_sh):
    cid = lax.axis_index("c")
    sid = lax.axis_index("s")
    wid = cid * NS + sid
    base = wid * CHE
    pltpu.sync_copy(ei_hbm.at[pl.ds(base, CHE)], src_v.at[pl.ds(0, CHE)])
    pltpu.sync_copy(ei_hbm.at[pl.ds(E + base, CHE)],
                    dst_v.at[pl.ds(0, CHE)])
    pltpu.sync_copy(dism_hbm, dism_v)

    def zb(r, _):
        for c in range(D // 16):
            zero_v[r, pl.ds(c * 16, 16)] = jnp.zeros((16,), _f32)
        return ()

    lax.fori_loop(0, ZROWS, zb, ())
    for k in range(ROWS_PER_TILE // ZROWS):
        pltpu.sync_copy(
            zero_v, agg_sh.at[pl.ds(sid * ROWS_PER_TILE + k * ZROWS, ZROWS)])
    plsc.subcore_barrier()

    def compact(cntv, s16, d16, m):
        idx = cntv + plsc.cumsum(m.astype(_i32)) - 1
        plsc.store_scatter(src_v, [idx], s16, mask=m)
        plsc.store_scatter(dst_v, [idx], d16, mask=m)
        return cntv + plsc.all_reduce_population_count(m)

    def edge_body(i, cntv):
        s16 = src_v[pl.ds(i * 16, 16)]
        d16 = dst_v[pl.ds(i * 16, 16)]
        m = plsc.load_gather(dism_v, [d16]) < 0.0
        return compact(cntv, s16, d16, m)

    cntv = lax.fori_loop(0, NCHUNK_E, edge_body, jnp.zeros((16,), _i32))

    nbase = wid * NLOOP

    def loop_body(i, cntv):
        v16 = nbase + i * 16 + lax.iota(_i32, 16)
        m = plsc.load_gather(dism_v, [v16]) < 0.0
        return compact(cntv, v16, v16, m)

    cntv = lax.fori_loop(0, NCHUNK_S, loop_body, cntv)
    cnt = jnp.max(cntv)

    tail = cnt + lax.iota(_i32, 16)
    plsc.store_scatter(src_v, [tail], jnp.zeros((16,), _i32))
    plsc.store_scatter(dst_v, [tail], jnp.full((16,), N, _i32))

    def proc_body(j, _):
        s16 = src_v[pl.ds(j * 16, 16)]
        d16 = dst_v[pl.ds(j * 16, 16)]
        dms = plsc.load_gather(dism_v, [s16])
        dmd = plsc.load_gather(dism_v, [d16])
        nrm = jnp.abs(dms) * jnp.abs(dmd)
        nrm_v[pl.ds(16, 16)] = nrm
        pltpu.sync_copy(x_hbm.at[s16], rows_v)
        for r in range(16):
            splat = plsc.load_gather(nrm_v, [jnp.full((16,), 16 + r, _i32)])
            for c in range(D // 16):
                rows_v[r, pl.ds(c * 16, 16)] = (
                    rows_v[r, pl.ds(c * 16, 16)] * splat)
        pltpu.sync_copy(rows_v, agg_sh.at[d16], add=True)
        return ()

    nch = (cnt + 15) // 16
    lax.fori_loop(0, nch, proc_body, ())
    plsc.subcore_barrier()
    pltpu.sync_copy(agg_sh.at[pl.ds(sid * ROWS_PER_TILE, ROWS_PER_TILE)],
                    agg_hbm.at[cid, pl.ds(sid * ROWS_PER_TILE, ROWS_PER_TILE)])


def _make_agg_kernel():
    mesh = plsc.VectorSubcoreMesh(core_axis_name="c", subcore_axis_name="s")
    return pl.kernel(
        _agg_body,
        out_type=jax.ShapeDtypeStruct((NC, NP, D), _f32),
        mesh=mesh,
        compiler_params=pltpu.CompilerParams(needs_layout_passes=False),
        scratch_types=[
            pltpu.VMEM((CAP,), _i32),
            pltpu.VMEM((CAP,), _i32),
            pltpu.VMEM((NP,), _f32),
            pltpu.VMEM((16, D), _f32),
            pltpu.VMEM((32,), _f32),
            pltpu.VMEM((ZROWS, D), _f32),
            pltpu.VMEM_SHARED((NP, D), _f32),
        ],
    )


_BLK = 1024
_NSTEP = NP // _BLK


def _final_body(p0_ref, p1_ref, w0_ref, w1_ref, b1_ref, w2_ref, b2_ref,
                out_ref, acc_ref):
    i = pl.program_id(0)

    @pl.when(i == 0)
    def _():
        acc_ref[...] = jnp.zeros_like(acc_ref)

    a = p0_ref[0] + p1_ref[0]
    h = a @ w1_ref[...] + b1_ref[...]
    h = jnp.maximum(h, 0.0)
    acc_ref[...] += lax.dot_general(
        w0_ref[...], h, (((0,), (0,)), ((), ())),
        preferred_element_type=_f32)

    @pl.when(i == _NSTEP - 1)
    def _():
        z = jnp.dot(acc_ref[...], w2_ref[...],
                    preferred_element_type=_f32) + b2_ref[...]
        mx = jnp.max(z, axis=1, keepdims=True)
        se = jnp.sum(jnp.exp(z - mx), axis=1, keepdims=True)
        out_ref[...] = z - mx - jnp.log(se)


def _final_kernel(agg, w0m, W1, b1, W2, b2):
    return pl.pallas_call(
        _final_body,
        grid=(_NSTEP,),
        in_specs=[
            pl.BlockSpec((1, _BLK, D), lambda i: (0, i, 0)),
            pl.BlockSpec((1, _BLK, D), lambda i: (1, i, 0)),
            pl.BlockSpec((_BLK, 8), lambda i: (i, 0)),
            pl.BlockSpec((D, H), lambda i: (0, 0)),
            pl.BlockSpec((1, H), lambda i: (0, 0)),
            pl.BlockSpec((H, C), lambda i: (0, 0)),
            pl.BlockSpec((1, C), lambda i: (0, 0)),
        ],
        out_specs=pl.BlockSpec((8, C), lambda i: (0, 0)),
        out_shape=jax.ShapeDtypeStruct((8, C), _f32),
        scratch_shapes=[pltpu.VMEM((8, H), _f32)],
    )(agg, agg, w0m, W1, b1, W2, b2)


def kernel(edge_index, x, W1, b1, W2, b2):
    ei = edge_index.astype(_i32).reshape(2 * E)
    hist = _make_hist_kernel()(ei)
    dw = _norm_kernel(hist)
    dism = dw[0].reshape(NP)
    w0 = dw[1].reshape(NP)

    agg = _make_agg_kernel()(ei, dism, x)

    w0m = jnp.zeros((NP, 8), _f32).at[:, 0].set(w0)
    res = _final_kernel(agg, w0m, W1, b1.reshape(1, H), W2, b2.reshape(1, C))
    return res[0]


# speedup vs baseline: 106.7702x; 1.0338x over previous
import functools

import jax
import jax.numpy as jnp
from jax import lax
from jax.experimental import pallas as pl
from jax.experimental.pallas import tpu as pltpu
from jax.experimental.pallas import tpu_sc as plsc

N = 10000
E = 320000
D = 128
H = 16
C = 10

NC = 2
NS = 16
NW = NC * NS

NP = 10240
NB = 2 * NP
CHE = E // NW
NCHUNK_E = CHE // 16
NLOOP = NP // NW
NCHUNK_S = NLOOP // 16
CAP = CHE + NLOOP + 16
COLS = NB // NS
ROWS_PER_TILE = NP // NS
ZROWS = 64

_f32 = jnp.float32
_i32 = jnp.int32


def _hist_body(ei_hbm, hist_hbm, src_v, dst_v, hist_v, red_v, acc_v, shared):
    cid = lax.axis_index("c")
    sid = lax.axis_index("s")
    wid = cid * NS + sid
    base = wid * CHE
    pltpu.sync_copy(ei_hbm.at[pl.ds(base, CHE)], src_v)
    pltpu.sync_copy(ei_hbm.at[pl.ds(E + base, CHE)], dst_v)

    def zero_body(i, _):
        hist_v[pl.ds(i * 16, 16)] = jnp.zeros((16,), _f32)
        return ()

    lax.fori_loop(0, NB // 16, zero_body, ())

    ones = jnp.ones((16,), _f32)

    @plsc.parallel_loop(0, CHE, step=16, unroll=4)
    def _(i):
        s16 = src_v[pl.ds(i, 16)]
        d16 = dst_v[pl.ds(i, 16)]
        plsc.addupdate_scatter(hist_v, [d16], ones)
        plsc.addupdate_scatter(hist_v, [s16 + NP], ones, mask=d16 == 0)

    pltpu.sync_copy(hist_v, shared.at[sid])
    plsc.subcore_barrier()
    colbase = sid * COLS
    pltpu.sync_copy(shared.at[:, pl.ds(colbase, COLS)], red_v)

    def red_body(j, _):
        acc = red_v[0, pl.ds(j * 16, 16)]
        for t in range(1, NS):
            acc = acc + red_v[t, pl.ds(j * 16, 16)]
        acc_v[pl.ds(j * 16, 16)] = acc
        return ()

    lax.fori_loop(0, COLS // 16, red_body, ())
    pltpu.sync_copy(acc_v, hist_hbm.at[cid, pl.ds(colbase, COLS)])


def _make_hist_kernel():
    mesh = plsc.VectorSubcoreMesh(core_axis_name="c", subcore_axis_name="s")
    return pl.kernel(
        _hist_body,
        out_type=jax.ShapeDtypeStruct((NC, NB), _f32),
        mesh=mesh,
        compiler_params=pltpu.CompilerParams(needs_layout_passes=False),
        scratch_types=[
            pltpu.VMEM((CHE,), _i32),
            pltpu.VMEM((CHE,), _i32),
            pltpu.VMEM((NB,), _f32),
            pltpu.VMEM((NS, COLS), _f32),
            pltpu.VMEM((COLS,), _f32),
            pltpu.VMEM_SHARED((NS, NB), _f32),
        ],
    )


def _norm_body(hist_ref, dw_ref):
    h = hist_ref[0] + hist_ref[1]
    row = lax.broadcasted_iota(_i32, (NP // 128, 128), 0)
    col = lax.broadcasted_iota(_i32, (NP // 128, 128), 1)
    at0 = jnp.logical_and(row == 0, col == 0).astype(_f32)
    deg = h[: NP // 128] + 1.0
    cnt0 = h[NP // 128:] + at0
    dis = lax.rsqrt(deg)
    dis0 = dis[0:1, 0:1]
    w0 = cnt0 * dis * dis0
    dw_ref[0] = jnp.where(w0 > 0.0, -dis, dis)
    dw_ref[1] = w0


def _norm_kernel(hist):
    hist3 = hist.reshape(NC, NB // 128, 128)
    return pl.pallas_call(
        _norm_body,
        out_shape=jax.ShapeDtypeStruct((2, NP // 128, 128), _f32),
    )(hist3)


def _agg_body(ei_hbm, dism_hbm, x_hbm, agg_hbm,
              src_v, dst_v, dism_v, rows_v, nrm_v, zero_v, agg_sh):
    cid = lax.axis_index("c")
    sid = lax.axis_index("s")
    wid = cid * NS + sid
    base = wid * CHE
    pltpu.sync_copy(ei_hbm.at[pl.ds(base, CHE)], src_v.at[pl.ds(0, CHE)])
    pltpu.sync_copy(ei_hbm.at[pl.ds(E + base, CHE)],
                    dst_v.at[pl.ds(0, CHE)])
    pltpu.sync_copy(dism_hbm, dism_v)

    def zb(r, _):
        for c in range(D // 16):
            zero_v[r, pl.ds(c * 16, 16)] = jnp.zeros((16,), _f32)
        return ()

    lax.fori_loop(0, ZROWS, zb, ())
    for k in range(ROWS_PER_TILE // ZROWS):
        pltpu.sync_copy(
            zero_v, agg_sh.at[pl.ds(sid * ROWS_PER_TILE + k * ZROWS, ZROWS)])
    plsc.subcore_barrier()

    def compact(cntv, s16, d16, m):
        idx = cntv + plsc.cumsum(m.astype(_i32)) - 1
        plsc.store_scatter(src_v, [idx], s16, mask=m)
        plsc.store_scatter(dst_v, [idx], d16, mask=m)
        return cntv + plsc.all_reduce_population_count(m)

    def edge_body(i, cntv):
        s16 = src_v[pl.ds(i * 16, 16)]
        d16 = dst_v[pl.ds(i * 16, 16)]
        m = plsc.load_gather(dism_v, [d16]) < 0.0
        return compact(cntv, s16, d16, m)

    cntv = lax.fori_loop(0, NCHUNK_E, edge_body, jnp.zeros((16,), _i32))

    nbase = wid * NLOOP

    def loop_body(i, cntv):
        v16 = nbase + i * 16 + lax.iota(_i32, 16)
        m = plsc.load_gather(dism_v, [v16]) < 0.0
        return compact(cntv, v16, v16, m)

    cntv = lax.fori_loop(0, NCHUNK_S, loop_body, cntv)
    cnt = jnp.max(cntv)

    tail = cnt + lax.iota(_i32, 16)
    plsc.store_scatter(src_v, [tail], jnp.zeros((16,), _i32))
    plsc.store_scatter(dst_v, [tail], jnp.full((16,), N, _i32))

    def proc_body(j, _):
        s16 = src_v[pl.ds(j * 16, 16)]
        d16 = dst_v[pl.ds(j * 16, 16)]
        dms = plsc.load_gather(dism_v, [s16])
        dmd = plsc.load_gather(dism_v, [d16])
        nrm = jnp.abs(dms) * jnp.abs(dmd)
        nrm_v[pl.ds(16, 16)] = nrm
        pltpu.sync_copy(x_hbm.at[s16], rows_v)
        for r in range(16):
            splat = plsc.load_gather(nrm_v, [jnp.full((16,), 16 + r, _i32)])
            for c in range(D // 16):
                rows_v[r, pl.ds(c * 16, 16)] = (
                    rows_v[r, pl.ds(c * 16, 16)] * splat)
        pltpu.sync_copy(rows_v, agg_sh.at[d16], add=True)
        return ()

    nch = (cnt + 15) // 16
    lax.fori_loop(0, nch, proc_body, ())
    plsc.subcore_barrier()
    pltpu.sync_copy(agg_sh.at[pl.ds(sid * ROWS_PER_TILE, ROWS_PER_TILE)],
                    agg_hbm.at[cid, pl.ds(sid * ROWS_PER_TILE, ROWS_PER_TILE)])


def _make_agg_kernel():
    mesh = plsc.VectorSubcoreMesh(core_axis_name="c", subcore_axis_name="s")
    return pl.kernel(
        _agg_body,
        out_type=jax.ShapeDtypeStruct((NC, NP, D), _f32),
        mesh=mesh,
        compiler_params=pltpu.CompilerParams(needs_layout_passes=False),
        scratch_types=[
            pltpu.VMEM((CAP,), _i32),
            pltpu.VMEM((CAP,), _i32),
            pltpu.VMEM((NP,), _f32),
            pltpu.VMEM((16, D), _f32),
            pltpu.VMEM((32,), _f32),
            pltpu.VMEM((ZROWS, D), _f32),
            pltpu.VMEM_SHARED((NP, D), _f32),
        ],
    )


_BLK = 1024
_NSTEP = NP // _BLK


def _final_body(p0_ref, p1_ref, w0_ref, w1_ref, b1_ref, w2_ref, b2_ref,
                out_ref, acc_ref):
    i = pl.program_id(0)

    @pl.when(i == 0)
    def _():
        acc_ref[...] = jnp.zeros_like(acc_ref)

    a = p0_ref[0] + p1_ref[0]
    h = a @ w1_ref[...] + b1_ref[...]
    h = jnp.maximum(h, 0.0)
    acc_ref[...] += lax.dot_general(
        w0_ref[...], h, (((0,), (0,)), ((), ())),
        preferred_element_type=_f32)

    @pl.when(i == _NSTEP - 1)
    def _():
        z = jnp.dot(acc_ref[...], w2_ref[...],
                    preferred_element_type=_f32) + b2_ref[...]
        mx = jnp.max(z, axis=1, keepdims=True)
        se = jnp.sum(jnp.exp(z - mx), axis=1, keepdims=True)
        out_ref[...] = z - mx - jnp.log(se)


def _final_kernel(agg, w0m, W1, b1, W2, b2):
    return pl.pallas_call(
        _final_body,
        grid=(_NSTEP,),
        in_specs=[
            pl.BlockSpec((1, _BLK, D), lambda i: (0, i, 0)),
            pl.BlockSpec((1, _BLK, D), lambda i: (1, i, 0)),
            pl.BlockSpec((_BLK, 8), lambda i: (i, 0)),
            pl.BlockSpec((D, H), lambda i: (0, 0)),
            pl.BlockSpec((1, H), lambda i: (0, 0)),
            pl.BlockSpec((H, C), lambda i: (0, 0)),
            pl.BlockSpec((1, C), lambda i: (0, 0)),
        ],
        out_specs=pl.BlockSpec((8, C), lambda i: (0, 0)),
        out_shape=jax.ShapeDtypeStruct((8, C), _f32),
        scratch_shapes=[pltpu.VMEM((8, H), _f32)],
    )(agg, agg, w0m, W1, b1, W2, b2)


def kernel(edge_index, x, W1, b1, W2, b2):
    ei = edge_index.astype(_i32).reshape(2 * E)
    hist = _make_hist_kernel()(ei)
    dw = _norm_kernel(hist)
    dism = dw[0].reshape(NP)
    w0 = dw[1].reshape(NP)

    agg = _make_agg_kernel()(ei, dism, x)

    w0m = jnp.zeros((NP, 8), _f32).at[:, 0].set(w0)
    res = _final_kernel(agg, w0m, W1, b1.reshape(1, H), W2, b2.reshape(1, C))
    return res[0]


# speedup vs baseline: 119.9620x; 1.1236x over previous
import functools

import jax
import jax.numpy as jnp
from jax import lax
from jax.experimental import pallas as pl
from jax.experimental.pallas import tpu as pltpu
from jax.experimental.pallas import tpu_sc as plsc

N = 10000
E = 320000
D = 128
H = 16
C = 10

NC = 2
NS = 16
NW = NC * NS

NP = 10240
NB = 2 * NP
CHE = E // NW
NCHUNK_E = CHE // 16
NLOOP = NP // NW
NCHUNK_S = NLOOP // 16
CAP = CHE + NLOOP + 16
COLS = NB // NS
ROWS_PER_TILE = NP // NS
ZROWS = 64

_f32 = jnp.float32
_i32 = jnp.int32


def _hist_body(ei_hbm, hist_hbm, src_v, dst_v, hist_v, red_v, acc_v, shared):
    cid = lax.axis_index("c")
    sid = lax.axis_index("s")
    wid = cid * NS + sid
    base = wid * CHE
    pltpu.sync_copy(ei_hbm.at[pl.ds(base, CHE)], src_v)
    pltpu.sync_copy(ei_hbm.at[pl.ds(E + base, CHE)], dst_v)

    def zero_body(i, _):
        hist_v[pl.ds(i * 16, 16)] = jnp.zeros((16,), _f32)
        return ()

    lax.fori_loop(0, NB // 16, zero_body, ())

    ones = jnp.ones((16,), _f32)

    @plsc.parallel_loop(0, CHE, step=16, unroll=4)
    def _(i):
        s16 = src_v[pl.ds(i, 16)]
        d16 = dst_v[pl.ds(i, 16)]
        plsc.addupdate_scatter(hist_v, [d16], ones)
        plsc.addupdate_scatter(hist_v, [s16 + NP], ones, mask=d16 == 0)

    pltpu.sync_copy(hist_v, shared.at[sid])
    plsc.subcore_barrier()
    colbase = sid * COLS
    pltpu.sync_copy(shared.at[:, pl.ds(colbase, COLS)], red_v)

    def red_body(j, _):
        acc = red_v[0, pl.ds(j * 16, 16)]
        for t in range(1, NS):
            acc = acc + red_v[t, pl.ds(j * 16, 16)]
        acc_v[pl.ds(j * 16, 16)] = acc
        return ()

    lax.fori_loop(0, COLS // 16, red_body, ())
    pltpu.sync_copy(acc_v, hist_hbm.at[cid, pl.ds(colbase, COLS)])


def _make_hist_kernel():
    mesh = plsc.VectorSubcoreMesh(core_axis_name="c", subcore_axis_name="s")
    return pl.kernel(
        _hist_body,
        out_type=jax.ShapeDtypeStruct((NC, NB), _f32),
        mesh=mesh,
        compiler_params=pltpu.CompilerParams(needs_layout_passes=False),
        scratch_types=[
            pltpu.VMEM((CHE,), _i32),
            pltpu.VMEM((CHE,), _i32),
            pltpu.VMEM((NB,), _f32),
            pltpu.VMEM((NS, COLS), _f32),
            pltpu.VMEM((COLS,), _f32),
            pltpu.VMEM_SHARED((NS, NB), _f32),
        ],
    )


def _norm_body(hist_ref, dw_ref):
    h = hist_ref[0] + hist_ref[1]
    row = lax.broadcasted_iota(_i32, (NP // 128, 128), 0)
    col = lax.broadcasted_iota(_i32, (NP // 128, 128), 1)
    at0 = jnp.logical_and(row == 0, col == 0).astype(_f32)
    deg = h[: NP // 128] + 1.0
    cnt0 = h[NP // 128:] + at0
    dis = lax.rsqrt(deg)
    dis0 = dis[0:1, 0:1]
    w0 = cnt0 * dis * dis0
    dw_ref[0] = jnp.where(w0 > 0.0, -dis, dis)
    dw_ref[1] = w0


def _norm_kernel(hist):
    hist3 = hist.reshape(NC, NB // 128, 128)
    return pl.pallas_call(
        _norm_body,
        out_shape=jax.ShapeDtypeStruct((2, NP // 128, 128), _f32),
    )(hist3)


def _agg_body(ei_hbm, dism_hbm, x_hbm, agg_hbm,
              src_v, dst_v, dism_v, rows_v, nrm_v, zero_v, agg_sh):
    cid = lax.axis_index("c")
    sid = lax.axis_index("s")
    wid = cid * NS + sid
    base = wid * CHE
    pltpu.sync_copy(ei_hbm.at[pl.ds(base, CHE)], src_v.at[pl.ds(0, CHE)])
    pltpu.sync_copy(ei_hbm.at[pl.ds(E + base, CHE)],
                    dst_v.at[pl.ds(0, CHE)])
    pltpu.sync_copy(dism_hbm, dism_v)

    def zb(r, _):
        for c in range(D // 16):
            zero_v[r, pl.ds(c * 16, 16)] = jnp.zeros((16,), _f32)
        return ()

    lax.fori_loop(0, ZROWS, zb, ())
    for k in range(ROWS_PER_TILE // ZROWS):
        pltpu.sync_copy(
            zero_v, agg_sh.at[pl.ds(sid * ROWS_PER_TILE + k * ZROWS, ZROWS)])
    plsc.subcore_barrier()

    def compact(cntv, s16, d16, m):
        idx = cntv + plsc.cumsum(m.astype(_i32)) - 1
        plsc.store_scatter(src_v, [idx], s16, mask=m)
        plsc.store_scatter(dst_v, [idx], d16, mask=m)
        return cntv + plsc.all_reduce_population_count(m)

    @plsc.parallel_loop(0, CHE, step=16, unroll=4,
                        carry=jnp.zeros((16,), _i32))
    def cntv(i, cntv):
        s16 = src_v[pl.ds(i, 16)]
        d16 = dst_v[pl.ds(i, 16)]
        m = plsc.load_gather(dism_v, [d16]) < 0.0
        return compact(cntv, s16, d16, m)

    nbase = wid * NLOOP

    def loop_body(i, cntv):
        v16 = nbase + i * 16 + lax.iota(_i32, 16)
        m = plsc.load_gather(dism_v, [v16]) < 0.0
        return compact(cntv, v16, v16, m)

    cntv = lax.fori_loop(0, NCHUNK_S, loop_body, cntv)
    cnt = jnp.max(cntv)

    tail = cnt + lax.iota(_i32, 16)
    plsc.store_scatter(src_v, [tail], jnp.zeros((16,), _i32))
    plsc.store_scatter(dst_v, [tail], jnp.full((16,), N, _i32))

    def proc_body(j, _):
        s16 = src_v[pl.ds(j * 16, 16)]
        d16 = dst_v[pl.ds(j * 16, 16)]
        dms = plsc.load_gather(dism_v, [s16])
        dmd = plsc.load_gather(dism_v, [d16])
        nrm = jnp.abs(dms) * jnp.abs(dmd)
        nrm_v[pl.ds(16, 16)] = nrm
        pltpu.sync_copy(x_hbm.at[s16], rows_v)
        for r in range(16):
            splat = plsc.load_gather(nrm_v, [jnp.full((16,), 16 + r, _i32)])
            for c in range(D // 16):
                rows_v[r, pl.ds(c * 16, 16)] = (
                    rows_v[r, pl.ds(c * 16, 16)] * splat)
        pltpu.sync_copy(rows_v, agg_sh.at[d16], add=True)
        return ()

    nch = (cnt + 15) // 16
    lax.fori_loop(0, nch, proc_body, ())
    plsc.subcore_barrier()
    pltpu.sync_copy(agg_sh.at[pl.ds(sid * ROWS_PER_TILE, ROWS_PER_TILE)],
                    agg_hbm.at[cid, pl.ds(sid * ROWS_PER_TILE, ROWS_PER_TILE)])


def _make_agg_kernel():
    mesh = plsc.VectorSubcoreMesh(core_axis_name="c", subcore_axis_name="s")
    return pl.kernel(
        _agg_body,
        out_type=jax.ShapeDtypeStruct((NC, NP, D), _f32),
        mesh=mesh,
        compiler_params=pltpu.CompilerParams(needs_layout_passes=False),
        scratch_types=[
            pltpu.VMEM((CAP,), _i32),
            pltpu.VMEM((CAP,), _i32),
            pltpu.VMEM((NP,), _f32),
            pltpu.VMEM((16, D), _f32),
            pltpu.VMEM((32,), _f32),
            pltpu.VMEM((ZROWS, D), _f32),
            pltpu.VMEM_SHARED((NP, D), _f32),
        ],
    )


_BLK = 1024
_NSTEP = NP // _BLK


def _final_body(p0_ref, p1_ref, w0_ref, w1_ref, b1_ref, w2_ref, b2_ref,
                out_ref, acc_ref):
    i = pl.program_id(0)

    @pl.when(i == 0)
    def _():
        acc_ref[...] = jnp.zeros_like(acc_ref)

    a = p0_ref[0] + p1_ref[0]
    h = a @ w1_ref[...] + b1_ref[...]
    h = jnp.maximum(h, 0.0)
    acc_ref[...] += lax.dot_general(
        w0_ref[...], h, (((0,), (0,)), ((), ())),
        preferred_element_type=_f32)

    @pl.when(i == _NSTEP - 1)
    def _():
        z = jnp.dot(acc_ref[...], w2_ref[...],
                    preferred_element_type=_f32) + b2_ref[...]
        mx = jnp.max(z, axis=1, keepdims=True)
        se = jnp.sum(jnp.exp(z - mx), axis=1, keepdims=True)
        out_ref[...] = z - mx - jnp.log(se)


def _final_kernel(agg, w0m, W1, b1, W2, b2):
    return pl.pallas_call(
        _final_body,
        grid=(_NSTEP,),
        in_specs=[
            pl.BlockSpec((1, _BLK, D), lambda i: (0, i, 0)),
            pl.BlockSpec((1, _BLK, D), lambda i: (1, i, 0)),
            pl.BlockSpec((_BLK, 8), lambda i: (i, 0)),
            pl.BlockSpec((D, H), lambda i: (0, 0)),
            pl.BlockSpec((1, H), lambda i: (0, 0)),
            pl.BlockSpec((H, C), lambda i: (0, 0)),
            pl.BlockSpec((1, C), lambda i: (0, 0)),
        ],
        out_specs=pl.BlockSpec((8, C), lambda i: (0, 0)),
        out_shape=jax.ShapeDtypeStruct((8, C), _f32),
        scratch_shapes=[pltpu.VMEM((8, H), _f32)],
    )(agg, agg, w0m, W1, b1, W2, b2)


def kernel(edge_index, x, W1, b1, W2, b2):
    ei = edge_index.astype(_i32).reshape(2 * E)
    hist = _make_hist_kernel()(ei)
    dw = _norm_kernel(hist)
    dism = dw[0].reshape(NP)
    w0 = dw[1].reshape(NP)

    agg = _make_agg_kernel()(ei, dism, x)

    w0m = jnp.zeros((NP, 8), _f32).at[:, 0].set(w0)
    res = _final_kernel(agg, w0m, W1, b1.reshape(1, H), W2, b2.reshape(1, C))
    return res[0]


# speedup vs baseline: 120.1995x; 1.0020x over previous
import functools

import jax
import jax.numpy as jnp
from jax import lax
from jax.experimental import pallas as pl
from jax.experimental.pallas import tpu as pltpu
from jax.experimental.pallas import tpu_sc as plsc

N = 10000
E = 320000
D = 128
H = 16
C = 10

NC = 2
NS = 16
NW = NC * NS

NP = 10240
NB = 2 * NP
CHE = E // NW
NCHUNK_E = CHE // 16
NLOOP = NP // NW
NCHUNK_S = NLOOP // 16
CAP = CHE + NLOOP + 16
COLS = NB // NS
ROWS_PER_TILE = NP // NS
ZROWS = 64

_f32 = jnp.float32
_i32 = jnp.int32


def _hist_body(ei_hbm, hist_hbm, src_v, dst_v, hist_v, red_v, acc_v, shared):
    cid = lax.axis_index("c")
    sid = lax.axis_index("s")
    wid = cid * NS + sid
    base = wid * CHE
    pltpu.sync_copy(ei_hbm.at[pl.ds(base, CHE)], src_v)
    pltpu.sync_copy(ei_hbm.at[pl.ds(E + base, CHE)], dst_v)

    def zero_body(i, _):
        hist_v[pl.ds(i * 16, 16)] = jnp.zeros((16,), _f32)
        return ()

    lax.fori_loop(0, NB // 16, zero_body, ())

    ones = jnp.ones((16,), _f32)

    @plsc.parallel_loop(0, CHE, step=16, unroll=8)
    def _(i):
        s16 = src_v[pl.ds(i, 16)]
        d16 = dst_v[pl.ds(i, 16)]
        plsc.addupdate_scatter(hist_v, [d16], ones)
        plsc.addupdate_scatter(hist_v, [s16 + NP], ones, mask=d16 == 0)

    pltpu.sync_copy(hist_v, shared.at[sid])
    plsc.subcore_barrier()
    colbase = sid * COLS
    pltpu.sync_copy(shared.at[:, pl.ds(colbase, COLS)], red_v)

    def red_body(j, _):
        acc = red_v[0, pl.ds(j * 16, 16)]
        for t in range(1, NS):
            acc = acc + red_v[t, pl.ds(j * 16, 16)]
        acc_v[pl.ds(j * 16, 16)] = acc
        return ()

    lax.fori_loop(0, COLS // 16, red_body, ())
    pltpu.sync_copy(acc_v, hist_hbm.at[cid, pl.ds(colbase, COLS)])


def _make_hist_kernel():
    mesh = plsc.VectorSubcoreMesh(core_axis_name="c", subcore_axis_name="s")
    return pl.kernel(
        _hist_body,
        out_type=jax.ShapeDtypeStruct((NC, NB), _f32),
        mesh=mesh,
        compiler_params=pltpu.CompilerParams(needs_layout_passes=False),
        scratch_types=[
            pltpu.VMEM((CHE,), _i32),
            pltpu.VMEM((CHE,), _i32),
            pltpu.VMEM((NB,), _f32),
            pltpu.VMEM((NS, COLS), _f32),
            pltpu.VMEM((COLS,), _f32),
            pltpu.VMEM_SHARED((NS, NB), _f32),
        ],
    )


def _norm_body(hist_ref, dw_ref):
    h = hist_ref[0] + hist_ref[1]
    row = lax.broadcasted_iota(_i32, (NP // 128, 128), 0)
    col = lax.broadcasted_iota(_i32, (NP // 128, 128), 1)
    at0 = jnp.logical_and(row == 0, col == 0).astype(_f32)
    deg = h[: NP // 128] + 1.0
    cnt0 = h[NP // 128:] + at0
    dis = lax.rsqrt(deg)
    dis0 = dis[0:1, 0:1]
    w0 = cnt0 * dis * dis0
    dw_ref[0] = jnp.where(w0 > 0.0, -dis, dis)
    dw_ref[1] = w0


def _norm_kernel(hist):
    hist3 = hist.reshape(NC, NB // 128, 128)
    return pl.pallas_call(
        _norm_body,
        out_shape=jax.ShapeDtypeStruct((2, NP // 128, 128), _f32),
    )(hist3)


def _agg_body(ei_hbm, dism_hbm, x_hbm, agg_hbm,
              src_v, dst_v, dism_v, rows_v, nrm_v, zero_v, agg_sh):
    cid = lax.axis_index("c")
    sid = lax.axis_index("s")
    wid = cid * NS + sid
    base = wid * CHE
    pltpu.sync_copy(ei_hbm.at[pl.ds(base, CHE)], src_v.at[pl.ds(0, CHE)])
    pltpu.sync_copy(ei_hbm.at[pl.ds(E + base, CHE)],
                    dst_v.at[pl.ds(0, CHE)])
    pltpu.sync_copy(dism_hbm, dism_v)

    def zb(r, _):
        for c in range(D // 16):
            zero_v[r, pl.ds(c * 16, 16)] = jnp.zeros((16,), _f32)
        return ()

    lax.fori_loop(0, ZROWS, zb, ())
    for k in range(ROWS_PER_TILE // ZROWS):
        pltpu.sync_copy(
            zero_v, agg_sh.at[pl.ds(sid * ROWS_PER_TILE + k * ZROWS, ZROWS)])
    plsc.subcore_barrier()

    def compact(cntv, s16, d16, m):
        idx = cntv + plsc.cumsum(m.astype(_i32)) - 1
        plsc.store_scatter(src_v, [idx], s16, mask=m)
        plsc.store_scatter(dst_v, [idx], d16, mask=m)
        return cntv + plsc.all_reduce_population_count(m)

    @plsc.parallel_loop(0, CHE, step=16, unroll=8,
                        carry=jnp.zeros((16,), _i32))
    def cntv(i, cntv):
        s16 = src_v[pl.ds(i, 16)]
        d16 = dst_v[pl.ds(i, 16)]
        m = plsc.load_gather(dism_v, [d16]) < 0.0
        return compact(cntv, s16, d16, m)

    nbase = wid * NLOOP

    def loop_body(i, cntv):
        v16 = nbase + i * 16 + lax.iota(_i32, 16)
        m = plsc.load_gather(dism_v, [v16]) < 0.0
        return compact(cntv, v16, v16, m)

    cntv = lax.fori_loop(0, NCHUNK_S, loop_body, cntv)
    cnt = jnp.max(cntv)

    tail = cnt + lax.iota(_i32, 16)
    plsc.store_scatter(src_v, [tail], jnp.zeros((16,), _i32))
    plsc.store_scatter(dst_v, [tail], jnp.full((16,), N, _i32))

    def proc_body(j, _):
        s16 = src_v[pl.ds(j * 16, 16)]
        d16 = dst_v[pl.ds(j * 16, 16)]
        dms = plsc.load_gather(dism_v, [s16])
        dmd = plsc.load_gather(dism_v, [d16])
        nrm = jnp.abs(dms) * jnp.abs(dmd)
        nrm_v[pl.ds(16, 16)] = nrm
        pltpu.sync_copy(x_hbm.at[s16], rows_v)
        for r in range(16):
            splat = plsc.load_gather(nrm_v, [jnp.full((16,), 16 + r, _i32)])
            for c in range(D // 16):
                rows_v[r, pl.ds(c * 16, 16)] = (
                    rows_v[r, pl.ds(c * 16, 16)] * splat)
        pltpu.sync_copy(rows_v, agg_sh.at[d16], add=True)
        return ()

    nch = (cnt + 15) // 16
    lax.fori_loop(0, nch, proc_body, ())
    plsc.subcore_barrier()
    pltpu.sync_copy(agg_sh.at[pl.ds(sid * ROWS_PER_TILE, ROWS_PER_TILE)],
                    agg_hbm.at[cid, pl.ds(sid * ROWS_PER_TILE, ROWS_PER_TILE)])


def _make_agg_kernel():
    mesh = plsc.VectorSubcoreMesh(core_axis_name="c", subcore_axis_name="s")
    return pl.kernel(
        _agg_body,
        out_type=jax.ShapeDtypeStruct((NC, NP, D), _f32),
        mesh=mesh,
        compiler_params=pltpu.CompilerParams(needs_layout_passes=False),
        scratch_types=[
            pltpu.VMEM((CAP,), _i32),
            pltpu.VMEM((CAP,), _i32),
            pltpu.VMEM((NP,), _f32),
            pltpu.VMEM((16, D), _f32),
            pltpu.VMEM((32,), _f32),
            pltpu.VMEM((ZROWS, D), _f32),
            pltpu.VMEM_SHARED((NP, D), _f32),
        ],
    )


_BLK = 1024
_NSTEP = NP // _BLK


def _final_body(p0_ref, p1_ref, w0_ref, w1_ref, b1_ref, w2_ref, b2_ref,
                out_ref, acc_ref):
    i = pl.program_id(0)

    @pl.when(i == 0)
    def _():
        acc_ref[...] = jnp.zeros_like(acc_ref)

    a = p0_ref[0] + p1_ref[0]
    h = a @ w1_ref[...] + b1_ref[...]
    h = jnp.maximum(h, 0.0)
    acc_ref[...] += lax.dot_general(
        w0_ref[...], h, (((0,), (0,)), ((), ())),
        preferred_element_type=_f32)

    @pl.when(i == _NSTEP - 1)
    def _():
        z = jnp.dot(acc_ref[...], w2_ref[...],
                    preferred_element_type=_f32) + b2_ref[...]
        mx = jnp.max(z, axis=1, keepdims=True)
        se = jnp.sum(jnp.exp(z - mx), axis=1, keepdims=True)
        out_ref[...] = z - mx - jnp.log(se)


def _final_kernel(agg, w0m, W1, b1, W2, b2):
    return pl.pallas_call(
        _final_body,
        grid=(_NSTEP,),
        in_specs=[
            pl.BlockSpec((1, _BLK, D), lambda i: (0, i, 0)),
            pl.BlockSpec((1, _BLK, D), lambda i: (1, i, 0)),
            pl.BlockSpec((_BLK, 8), lambda i: (i, 0)),
            pl.BlockSpec((D, H), lambda i: (0, 0)),
            pl.BlockSpec((1, H), lambda i: (0, 0)),
            pl.BlockSpec((H, C), lambda i: (0, 0)),
            pl.BlockSpec((1, C), lambda i: (0, 0)),
        ],
        out_specs=pl.BlockSpec((8, C), lambda i: (0, 0)),
        out_shape=jax.ShapeDtypeStruct((8, C), _f32),
        scratch_shapes=[pltpu.VMEM((8, H), _f32)],
    )(agg, agg, w0m, W1, b1, W2, b2)


def kernel(edge_index, x, W1, b1, W2, b2):
    ei = edge_index.astype(_i32).reshape(2 * E)
    hist = _make_hist_kernel()(ei)
    dw = _norm_kernel(hist)
    dism = dw[0].reshape(NP)
    w0 = dw[1].reshape(NP)

    agg = _make_agg_kernel()(ei, dism, x)

    w0m = jnp.zeros((NP, 8), _f32).at[:, 0].set(w0)
    res = _final_kernel(agg, w0m, W1, b1.reshape(1, H), W2, b2.reshape(1, C))
    return res[0]
